# baseline (speedup 1.0000x reference)
import functools

import jax
import jax.numpy as jnp
from jax import lax
from jax.experimental import pallas as pl
from jax.experimental.pallas import tpu as pltpu
from jax.experimental.pallas import tpu_sc as plsc


def kernel(x, input_ids, W):
    B = x.shape[0]
    L = input_ids.shape[0]
    D = W.shape[1]
    info = plsc.get_sparse_core_info()
    NC, NS = info.num_cores, info.num_subcores
    NW = NC * NS
    rpt = L // NS
    bpw = B // NW
    ids2 = input_ids.reshape(NS, rpt)
    mesh = plsc.VectorSubcoreMesh(core_axis_name="c", subcore_axis_name="s")

    @functools.partial(
        pl.kernel,
        mesh=mesh,
        out_type=jax.ShapeDtypeStruct((B, L, D), jnp.float32),
        scratch_types=[
            pltpu.VMEM((rpt,), jnp.int32),
            pltpu.VMEM((rpt, D), jnp.float32),
            pltpu.VMEM_SHARED((L, D), jnp.float32),
            pltpu.SemaphoreType.DMA,
            pltpu.SemaphoreType.DMA,
        ],
    )
    def emb_kernel(ids_hbm, table_hbm, out_hbm, idx_v, rows_v, emb_sh,
                   gsem, wsem):
        c = lax.axis_index("c")
        s = lax.axis_index("s")
        pltpu.sync_copy(ids_hbm.at[s], idx_v)
        pltpu.async_copy(table_hbm.at[idx_v], rows_v, gsem).wait()
        pltpu.sync_copy(rows_v, emb_sh.at[pl.ds(s * rpt, rpt)])
        plsc.subcore_barrier()
        wid = s * NC + c
        copies = [
            pltpu.async_copy(emb_sh, out_hbm.at[wid * bpw + j], wsem)
            for j in range(bpw)
        ]
        for cp in copies:
            cp.wait()

    return emb_kernel(ids2, W)


# speedup vs baseline: 1.1458x; 1.1458x over previous
import functools

import jax
import jax.numpy as jnp
from jax import lax
from jax.experimental import pallas as pl
from jax.experimental.pallas import tpu as pltpu
from jax.experimental.pallas import tpu_sc as plsc


def kernel(x, input_ids, W):
    B = x.shape[0]
    L = input_ids.shape[0]
    D = W.shape[1]
    info = plsc.get_sparse_core_info()
    NC = info.num_cores
    NW = NC * info.num_subcores
    rpw = L // NW
    ids2 = input_ids.reshape(NW, rpw)
    mesh = plsc.VectorSubcoreMesh(core_axis_name="c", subcore_axis_name="s")

    @functools.partial(
        pl.kernel,
        mesh=mesh,
        out_type=jax.ShapeDtypeStruct((B, L, D), jnp.float32),
        scratch_types=[
            pltpu.VMEM((rpw,), jnp.int32),
            pltpu.VMEM((rpw, D), jnp.float32),
            pltpu.SemaphoreType.DMA,
            pltpu.SemaphoreType.DMA,
        ],
    )
    def emb_kernel(ids_hbm, table_hbm, out_hbm, idx_v, rows_v, gsem, wsem):
        wid = lax.axis_index("s") * NC + lax.axis_index("c")
        base = wid * rpw
        pltpu.sync_copy(ids_hbm.at[wid], idx_v)
        pltpu.async_copy(table_hbm.at[idx_v], rows_v, gsem).wait()
        copies = [
            pltpu.async_copy(rows_v, out_hbm.at[b, pl.ds(base, rpw)], wsem)
            for b in range(B)
        ]
        for c in copies:
            c.wait()

    return emb_kernel(ids2, W)
